# baseline (speedup 1.0000x reference)
import functools

import jax
import jax.numpy as jnp
from jax import lax
from jax.experimental import pallas as pl
from jax.experimental.pallas import tpu as pltpu
from jax.experimental.pallas import tpu_sc as plsc

NC, NS = 2, 16
NW = NC * NS
D = 128
DH = D // 2
CHUNK = 128
ZROWS = 128
NBUF = 4


def _make_seg_sum(n_edges_pad, n_dst_pad, n_src_pad):
    epw = n_edges_pad // NS
    n_chunks = epw // CHUNK
    assert n_chunks % NBUF == 0 and n_chunks >= 3 * NBUF
    rpt = n_dst_pad // NS
    nz = rpt // ZROWS

    mesh = plsc.VectorSubcoreMesh(
        core_axis_name="c", subcore_axis_name="s",
        num_cores=NC, num_subcores=NS)

    idx_t = [pltpu.VMEM((CHUNK,), jnp.int32) for _ in range(2 * NBUF)]
    rows_t = [pltpu.VMEM((CHUNK, DH), jnp.float32) for _ in range(NBUF)]
    sem_t = [pltpu.SemaphoreType.DMA for _ in range(3 * NBUF)]

    @functools.partial(
        pl.kernel,
        out_type=(
            jax.ShapeDtypeStruct((NC * n_dst_pad, DH), jnp.float32),
            jax.ShapeDtypeStruct((NW, n_dst_pad), jnp.float32),
        ),
        mesh=mesh,
        compiler_params=pltpu.CompilerParams(needs_layout_passes=False, use_tc_tiling_on_sc=False),
        scratch_types=[
            pltpu.VMEM((ZROWS, DH), jnp.float32),
            pltpu.VMEM((n_dst_pad,), jnp.float32),
            pltpu.VMEM_SHARED((n_dst_pad, DH), jnp.float32),
        ] + idx_t + rows_t + sem_t,
    )
    def seg_sum(x2_hbm, src_hbm, dst_hbm, z_hbm, zcnt_hbm,
                sum_out, cnt_out, zf_v, cnt_v, sh_sum, *bufs):
        is_v = bufs[0:NBUF]
        id_v = bufs[NBUF:2 * NBUF]
        rows_v = bufs[2 * NBUF:3 * NBUF]
        sem_i = bufs[3 * NBUF:4 * NBUF]
        sem_g = bufs[4 * NBUF:5 * NBUF]
        sem_s = bufs[5 * NBUF:6 * NBUF]

        cid = lax.axis_index("c")
        sid = lax.axis_index("s")

        pltpu.sync_copy(z_hbm, zf_v)
        pltpu.sync_copy(zcnt_hbm, cnt_v)
        for k in range(nz):
            pltpu.sync_copy(zf_v, sh_sum.at[pl.ds(sid * rpt + k * ZROWS,
                                                  ZROWS)])
        plsc.subcore_barrier()

        base = sid * epw
        src_off = cid * n_src_pad
        ones16 = jnp.full((16,), 1.0, jnp.float32) * (cid == 0).astype(
            jnp.float32)

        def issue_idx(j, b):
            off = base + j * CHUNK
            pltpu.async_copy(src_hbm.at[pl.ds(off, CHUNK)], is_v[b], sem_i[b])
            pltpu.async_copy(dst_hbm.at[pl.ds(off, CHUNK)], id_v[b], sem_i[b])

        def wait_idx(b):
            pltpu.make_async_copy(
                src_hbm.at[pl.ds(0, CHUNK)], is_v[b], sem_i[b]).wait()
            pltpu.make_async_copy(
                dst_hbm.at[pl.ds(0, CHUNK)], id_v[b], sem_i[b]).wait()

        def shift_src(b):
            for j in range(CHUNK // 16):
                sl = pl.ds(j * 16, 16)
                is_v[b][sl] = is_v[b][sl] + src_off

        def issue_gather(b):
            pltpu.async_copy(x2_hbm.at[is_v[b]], rows_v[b], sem_g[b])

        def wait_gather(b):
            pltpu.make_async_copy(x2_hbm.at[is_v[b]], rows_v[b],
                                  sem_g[b]).wait()

        def issue_scatter(b):
            pltpu.async_copy(rows_v[b], sh_sum.at[id_v[b]], sem_s[b],
                             add=True)

        def wait_scatter(b):
            pltpu.make_async_copy(rows_v[b], sh_sum.at[id_v[b]],
                                  sem_s[b]).wait()

        def count(b):
            for j in range(CHUNK // 16):
                dvec = id_v[b][pl.ds(j * 16, 16)]
                plsc.addupdate_scatter(cnt_v, [dvec], ones16)

        def pipe_iter(j, b):
            n1, n2 = (b + 1) % NBUF, (b + 2) % NBUF
            wait_idx(n1)
            shift_src(n1)
            issue_gather(n1)
            wait_scatter(n2)
            issue_idx(jnp.minimum(j + 2, n_chunks - 1), n2)
            wait_gather(b)
            issue_scatter(b)
            count(b)

        issue_idx(0, 0)
        issue_idx(1, 1)
        wait_idx(0)
        shift_src(0)
        issue_gather(0)
        pltpu.sync_copy(z_hbm, rows_v[2])
        pltpu.sync_copy(z_hbm, rows_v[3])
        pltpu.sync_copy(dst_hbm.at[pl.ds(base, CHUNK)], id_v[2])
        pltpu.sync_copy(dst_hbm.at[pl.ds(base, CHUNK)], id_v[3])
        issue_scatter(2)
        issue_scatter(3)

        def body(g, carry):
            for b in range(NBUF):
                pipe_iter(g * NBUF + b, b)
            return carry

        lax.fori_loop(0, n_chunks // NBUF, body, 0)

        wait_gather(0)
        wait_idx(1)
        wait_scatter(2)
        wait_scatter(3)

        plsc.subcore_barrier()

        obase = cid * n_dst_pad + sid * rpt
        pltpu.sync_copy(sh_sum.at[pl.ds(sid * rpt, rpt)],
                        sum_out.at[pl.ds(obase, rpt)])
        wid = sid * NC + cid
        pltpu.sync_copy(cnt_v, cnt_out.at[wid])

    return seg_sum


def _tc_body(act, sum0, sum1, cnt, xr, wl, wr, br, o):
    s = jnp.concatenate([sum0[...], sum1[...]], axis=1)
    c = jnp.maximum(jnp.sum(cnt[...], axis=0), 1.0)
    agg = s / c[:, None]
    y = (lax.dot_general(agg, wl[...], (((1,), (1,)), ((), ())),
                         preferred_element_type=jnp.float32)
         + lax.dot_general(xr[...], wr[...], (((1,), (1,)), ((), ())),
                           preferred_element_type=jnp.float32)
         + br[...])
    o[...] = act(y)


def _make_dense(n_dst_pad, act):
    B = 1024
    grid = n_dst_pad // B
    nblk = grid

    def call(sum_flat, cnt_parts, x_dst, wl, wr, b2d):
        return pl.pallas_call(
            functools.partial(_tc_body, act),
            grid=(grid,),
            in_specs=[
                pl.BlockSpec((B, DH), lambda i: (i, 0)),
                pl.BlockSpec((B, DH), lambda i: (i + nblk, 0)),
                pl.BlockSpec((NW, B), lambda i: (0, i)),
                pl.BlockSpec((B, D), lambda i: (i, 0)),
                pl.BlockSpec((D, D), lambda i: (0, 0)),
                pl.BlockSpec((D, D), lambda i: (0, 0)),
                pl.BlockSpec((1, D), lambda i: (0, 0)),
            ],
            out_specs=pl.BlockSpec((B, D), lambda i: (i, 0)),
            out_shape=jax.ShapeDtypeStruct((n_dst_pad, D), jnp.float32),
        )(sum_flat, sum_flat, cnt_parts, x_dst, wl, wr, b2d)

    return call


N0, N1, N2 = 50000, 10000, 2000
E1, E2 = 320000, 64000
P1, P2 = 10240, 2048
XP1, XP2 = 10240, 10240
E1P = 160 * CHUNK * NS
E2P = 32 * CHUNK * NS

_seg1 = _make_seg_sum(E1P, P1, XP1)
_seg2 = _make_seg_sum(E2P, P2, XP2)
_dense1 = _make_dense(P1, jax.nn.relu)
_dense2 = _make_dense(P2, jax.nn.sigmoid)


def _pad_edges(ei, e_pad, dst_pad_row):
    pad = jnp.tile(jnp.array([[0], [dst_pad_row]], jnp.int32),
                   (1, e_pad - ei.shape[1]))
    return jnp.concatenate([ei, pad], axis=1)


def _stack_halves(xs):
    return jnp.concatenate([xs[:, :DH], xs[:, DH:]], axis=0)


def kernel(x, edge_index1, edge_index2, W1l, W1r, b1, W2l, W2r, b2):
    z = jnp.zeros((ZROWS, DH), jnp.float32)
    zc1 = jnp.zeros((P1,), jnp.float32)
    zc2 = jnp.zeros((P2,), jnp.float32)
    ei1 = _pad_edges(edge_index1, E1P, P1 - 1)
    ei2 = _pad_edges(edge_index2, E2P, P2 - 1)

    x2 = _stack_halves(x[:XP1])
    sum1, cnt1 = _seg1(x2, ei1[0], ei1[1], z, zc1)
    h = _dense1(sum1, cnt1, x, W1l, W1r, b1.reshape(1, D))
    h2 = _stack_halves(h)
    sum2, cnt2 = _seg2(h2, ei2[0], ei2[1], z, zc2)
    out = _dense2(sum2, cnt2, h, W2l, W2r, b2.reshape(1, D))
    return out[:N2]
